# Initial kernel scaffold; baseline (speedup 1.0000x reference)
import jax
import jax.numpy as jnp
from jax.experimental import pallas as pl


def kernel(X, X_id, Z, P, alpha, beta):
    raise NotImplementedError("write your pallas kernel here")


# speedup vs baseline: 1.0461x; 1.0461x over previous
import functools

import jax
import jax.numpy as jnp
from jax import lax
from jax.experimental import pallas as pl
from jax.experimental.pallas import tpu as pltpu
from jax.experimental.pallas import tpu_sc as plsc

B = 16384
NC = 2
NS = 16
NW = NC * NS
CHUNK = B // NW
L = 16
GSLICE = 128
NG = CHUNK // GSLICE


def _tsbrnn_body(x_hbm, xid_hbm, z_hbm, p_hbm, alpha_hbm, beta_hbm,
                 y_hbm, zn_hbm, pn_hbm,
                 idx_v, a_v, b_v, x_v, z_v, p_v, y_v, zn_v, pn_v, sem):
    wid = lax.axis_index("s") * NC + lax.axis_index("c")
    base = wid * CHUNK

    pltpu.sync_copy(xid_hbm.at[pl.ds(base, CHUNK)], idx_v)
    copies = []
    for g in range(NG):
        sl = pl.ds(g * GSLICE, GSLICE)
        copies.append(pltpu.async_copy(alpha_hbm.at[idx_v.at[sl]], a_v.at[sl], sem))
        copies.append(pltpu.async_copy(beta_hbm.at[idx_v.at[sl]], b_v.at[sl], sem))
    pltpu.sync_copy(x_hbm.at[pl.ds(base, CHUNK)], x_v)
    pltpu.sync_copy(z_hbm.at[pl.ds(base, CHUNK)], z_v)
    pltpu.sync_copy(p_hbm.at[pl.ds(base, CHUNK)], p_v)
    for cp in copies:
        cp.wait()

    def step(i, carry):
        sl = pl.ds(i * L, L)
        x = x_v[sl]
        z = z_v[sl]
        p = p_v[sl]
        a = a_v[sl]
        b = b_v[sl]
        nz = x != 0.0
        zn = jnp.where(nz, a * x + (1.0 - a) * z, z)
        pn = jnp.where(nz, b, 0.0) + (1.0 - b) * p
        y_v[sl] = zn * pn
        zn_v[sl] = zn
        pn_v[sl] = pn
        return carry

    lax.fori_loop(0, CHUNK // L, step, 0)

    pltpu.sync_copy(y_v, y_hbm.at[pl.ds(base, CHUNK)])
    pltpu.sync_copy(zn_v, zn_hbm.at[pl.ds(base, CHUNK)])
    pltpu.sync_copy(pn_v, pn_hbm.at[pl.ds(base, CHUNK)])


@jax.jit
def _tsbrnn(x, xid, z, p, alpha, beta):
    mesh = plsc.VectorSubcoreMesh(
        core_axis_name="c", subcore_axis_name="s",
        num_cores=NC, num_subcores=NS)
    vec = jax.ShapeDtypeStruct((B,), jnp.float32)
    run = pl.kernel(
        _tsbrnn_body,
        out_type=(vec, vec, vec),
        mesh=mesh,
        scratch_types=[
            pltpu.VMEM((CHUNK,), jnp.int32),
            pltpu.VMEM((CHUNK,), jnp.float32),
            pltpu.VMEM((CHUNK,), jnp.float32),
            pltpu.VMEM((CHUNK,), jnp.float32),
            pltpu.VMEM((CHUNK,), jnp.float32),
            pltpu.VMEM((CHUNK,), jnp.float32),
            pltpu.VMEM((CHUNK,), jnp.float32),
            pltpu.VMEM((CHUNK,), jnp.float32),
            pltpu.VMEM((CHUNK,), jnp.float32),
            pltpu.SemaphoreType.DMA,
        ],
    )
    return run(x, xid, z, p, alpha, beta)


def kernel(X, X_id, Z, P, alpha, beta):
    y, zn, pn = _tsbrnn(X[:, 0], X_id[:, 0], Z[:, 0], P[:, 0],
                        alpha[:, 0], beta[:, 0])
    shp = X.shape
    return (y.reshape(shp), zn.reshape(shp), pn.reshape(shp))
